# baseline (speedup 1.0000x reference)
import functools

import jax
import jax.numpy as jnp
from jax import lax
from jax.experimental import pallas as pl

_N = 4096
_D = 256
_H = 4
_NRB = 2

_BM = 512
_BN = 512


def _decoder_body(hi_ref, hj_ref, t_ref, preds_ref, loss_ref):
    i = pl.program_id(0)
    j = pl.program_id(1)
    z = lax.dot_general(
        hi_ref[...], hj_ref[...], (((1,), (1,)), ((), ())),
        preferred_element_type=jnp.float32)
    p = 1.0 / (1.0 + jnp.exp(-z))
    logp = jnp.maximum(jnp.log(p), -100.0)
    log1mp = jnp.maximum(jnp.log(1.0 - p), -100.0)
    t = t_ref[...]
    contrib = -(t * logp + (1.0 - t) * log1mp)
    partial = jnp.sum(contrib, axis=(0, 1), keepdims=True) * (1.0 / (_N * _N))
    preds_ref[...] = p

    @pl.when((i == 0) & (j == 0))
    def _init():
        loss_ref[...] = jnp.zeros_like(loss_ref)

    loss_ref[...] += partial


def _decode(h, targets):
    grid = (_N // _BM, _N // _BN)
    preds, loss = pl.pallas_call(
        _decoder_body,
        grid=grid,
        in_specs=[
            pl.BlockSpec((_BM, _D), lambda i, j: (i, 0)),
            pl.BlockSpec((_BN, _D), lambda i, j: (j, 0)),
            pl.BlockSpec((_BM, _BN), lambda i, j: (i, j)),
        ],
        out_specs=[
            pl.BlockSpec((_BM, _BN), lambda i, j: (i, j)),
            pl.BlockSpec((1, 1), lambda i, j: (0, 0)),
        ],
        out_shape=[
            jax.ShapeDtypeStruct((_N, _N), jnp.float32),
            jax.ShapeDtypeStruct((1, 1), jnp.float32),
        ],
    )(h, h, targets)
    return loss[0, 0], preds


def _wgcn_enc(x, src, dst, ew, W, b):
    m = x[src] * ew[:, None]
    agg = jax.ops.segment_sum(m, dst, num_segments=_N)
    return jax.nn.relu(agg @ W + b)


def _gat_enc(x, src, dst, W, al, ar, bias):
    feat = (x @ W).reshape(_N, _H, -1)
    el = (feat * al[None, :, :]).sum(-1)
    er = (feat * ar[None, :, :]).sum(-1)
    e = jax.nn.leaky_relu(el[src] + er[dst], 0.2)
    emax = jax.ops.segment_max(e, dst, num_segments=_N)
    emax = jnp.where(jnp.isfinite(emax), emax, 0.0)
    ee = jnp.exp(e - emax[dst])
    den = jax.ops.segment_sum(ee, dst, num_segments=_N)
    alpha = ee / den[dst]
    out = jax.ops.segment_sum(alpha[:, :, None] * feat[src], dst,
                              num_segments=_N)
    return out + bias.reshape(1, _H, -1)


def kernel(feature, edge_index_f, edge_index_b, e_f, e_b, targets, W_f, b_f,
           W_b, b_b, W_id, b_id, W1, b1, gat_W, gat_al, gat_ar, gat_b,
           gamma, beta):
    sf, df = edge_index_f[0], edge_index_f[1]
    sb, db = edge_index_b[0], edge_index_b[1]

    Hid = feature @ W_id + b_id
    fH = _wgcn_enc(feature, sf, df, e_f, W_f, b_f)
    bH = _wgcn_enc(feature, sb, db, e_b, W_b, b_b)
    hidden = Hid + fH + bH
    hidden = feature + hidden * jax.nn.sigmoid(feature)
    hidden = hidden @ W1 + b1

    idx = 0
    for _blk in range(_NRB):
        x = hidden
        for _lyr in range(2):
            of = _gat_enc(x, sf, df, gat_W[idx], gat_al[idx], gat_ar[idx],
                          gat_b[idx]); idx += 1
            ob = _gat_enc(x, sb, db, gat_W[idx], gat_al[idx], gat_ar[idx],
                          gat_b[idx]); idx += 1
            x = (of + ob).mean(1)
        hidden = x + hidden

    mu = hidden.mean(0)
    var = ((hidden - mu) ** 2).mean(0)
    h = (hidden - mu) / jnp.sqrt(var + 1e-5) * gamma + beta

    loss, preds = _decode(h, targets)
    return (loss, preds)


# speedup vs baseline: 3.6382x; 3.6382x over previous
import functools

import jax
import jax.numpy as jnp
from jax import lax
from jax.experimental import pallas as pl
from jax.experimental.pallas import tpu as pltpu
from jax.experimental.pallas import tpu_sc as plsc

_N = 4096
_D = 256
_H = 4
_NRB = 2
_E = 131072

_BM = 512
_BN = 512


_NC = 2
_NS = 16
_NW = _NC * _NS
_RPT = _N // _NW
_EPT = _E // _NW
_KE = 128
_CAPG = 66
_CAP = _CAPG * _KE
_CHUNK = 2048


def _part_body(dst_hbm, ids_hbm, ldst_hbm, ngrp_hbm,
               chunk_v, ids_v, ldst_v, n_v):
    c = lax.axis_index("c")
    s = lax.axis_index("s")
    wid = s * _NC + c
    lo = wid * _RPT
    iota = lax.iota(jnp.int32, 16)

    sent_v = jnp.full((16,), _RPT << 17, jnp.int32)

    def chunk_loop(i, state):
        cursor0, p0, pending0 = state
        cursor0 = jnp.minimum(cursor0, _CAP - _KE - 2 * _CHUNK)
        pltpu.sync_copy(dst_hbm.at[pl.ds(i * _CHUNK, _CHUNK)], chunk_v)

        def vec_loop(v, st):
            cursor, p, pending = st
            d16 = chunk_v[pl.ds(v * 16, 16)]
            rel = d16 - lo
            m = (rel >= 0) & (rel < _RPT)
            mi = m.astype(jnp.int32)
            eid = iota + (i * _CHUNK + v * 16)
            pk = jnp.where(m, eid | (rel << 17), _RPT << 17)
            carry = sent_v
            ps = p
            for r in range(16):
                mr = mi[r] > 0
                s_r = pk[r]
                pending = jnp.where((iota == ps) & mr, s_r, pending)
                carry = jnp.where((iota == (ps - 16)) & mr, s_r, carry)
                ps = ps + mi[r]
            ids_v[pl.ds(cursor, 16)] = pending
            ids_v[pl.ds(cursor + 16, 16)] = carry
            full = ps >= 16
            adv = jnp.where(full, 16, 0)
            cursor = cursor + adv
            p = ps - adv
            pending = jnp.where(jnp.broadcast_to(full, (16,)), carry, pending)
            return (cursor, p, pending)

        return lax.fori_loop(0, _CHUNK // 16, vec_loop,
                             (cursor0, p0, pending0))

    cursor, p, pending = lax.fori_loop(
        0, _E // _CHUNK, chunk_loop,
        (jnp.int32(0), jnp.int32(0), sent_v))
    ids_v[pl.ds(cursor, 16)] = pending
    for j in range(1, (_KE // 16) + 2):
        ids_v[pl.ds(cursor + j * 16, 16)] = sent_v
    cnt = cursor + p
    ng = (cnt + _KE - 1) // _KE
    ng = jnp.maximum(ng, 1)
    n_v[...] = jnp.broadcast_to(ng, (16,)).astype(jnp.int32)
    pltpu.sync_copy(n_v, ngrp_hbm.at[wid])

    def unpack(u, carry):
        pk = ids_v[pl.ds(u * 16, 16)]
        ldst_v[pl.ds(u * 16, 16)] = pk >> 17
        ids_v[pl.ds(u * 16, 16)] = pk & 0x1FFFF
        return carry

    lax.fori_loop(0, _CAP // 16, unpack, 0)
    pltpu.sync_copy(ids_v, ids_hbm.at[wid])
    pltpu.sync_copy(ldst_v, ldst_hbm.at[wid])


@jax.jit
def _sc_partition(dst):
    k = pl.kernel(
        _part_body,
        mesh=plsc.VectorSubcoreMesh(core_axis_name="c", subcore_axis_name="s"),
        out_type=[
            jax.ShapeDtypeStruct((_NW, _CAP), jnp.int32),
            jax.ShapeDtypeStruct((_NW, _CAP), jnp.int32),
            jax.ShapeDtypeStruct((_NW, 16), jnp.int32),
        ],
        scratch_types=[
            pltpu.VMEM((_CHUNK,), jnp.int32),
            pltpu.VMEM((_CAP,), jnp.int32),
            pltpu.VMEM((_CAP,), jnp.int32),
            pltpu.VMEM((16,), jnp.int32),
        ],
    )
    ids, ldst, ngrp = k(dst)
    return (ids.reshape(_NW, _CAPG, _KE), ldst.reshape(_NW, _CAPG, _KE), ngrp)


def _agg_body(feat_hbm, alpha_hbm, ids_hbm, ldst_hbm, ngrp_hbm, out_hbm,
              ids_v, ldst_v, alpha_g, rows_v, n_v, acc, sem):
    c = lax.axis_index("c")
    s = lax.axis_index("s")
    wid = s * _NC + c
    pltpu.sync_copy(ids_hbm.at[wid], ids_v)
    pltpu.sync_copy(ldst_hbm.at[wid], ldst_v)
    pltpu.sync_copy(ngrp_hbm.at[wid], n_v)
    ng = n_v[...][0]

    zero = jnp.zeros((16,), jnp.float32)

    def zrow(r, carry):
        for v in range(_D // 16):
            acc[r, pl.ds(v * 16, 16)] = zero
        return carry

    lax.fori_loop(0, _RPT + 8, zrow, 0)

    def group(g, carry):
        cp1 = pltpu.async_copy(feat_hbm.at[ids_v.at[g]], rows_v, sem)
        cp2 = pltpu.async_copy(alpha_hbm.at[ids_v.at[g]], alpha_g, sem)
        cp1.wait()
        cp2.wait()

        def sub(kk, carry2):
            a16 = alpha_g[pl.ds(kk * 16, 16)]
            l16 = ldst_v[g, pl.ds(kk * 16, 16)]
            rbase = kk * 16
            for r in range(16):
                a = a16[r]
                lr = l16[r]
                for v in range(_D // 16):
                    acc[lr, pl.ds(v * 16, 16)] = (
                        acc[lr, pl.ds(v * 16, 16)]
                        + rows_v[rbase + r, pl.ds(v * 16, 16)] * a)
            return carry2

        lax.fori_loop(0, _KE // 16, sub, 0)
        return carry

    lax.fori_loop(0, ng, group, 0)
    pltpu.sync_copy(acc.at[pl.ds(0, _RPT)],
                    out_hbm.at[pl.ds(wid * _RPT, _RPT)])


@jax.jit
def _sc_agg(feat, alpha, part):
    ids, ldst, ngrp = part
    k = pl.kernel(
        _agg_body,
        mesh=plsc.VectorSubcoreMesh(core_axis_name="c", subcore_axis_name="s"),
        out_type=jax.ShapeDtypeStruct((_N, _D), jnp.float32),
        scratch_types=[
            pltpu.VMEM((_CAPG, _KE), jnp.int32),
            pltpu.VMEM((_CAPG, _KE), jnp.int32),
            pltpu.VMEM((_KE,), jnp.float32),
            pltpu.VMEM((_KE, _D), jnp.float32),
            pltpu.VMEM((16,), jnp.int32),
            pltpu.VMEM((_RPT + 8, _D), jnp.float32),
            pltpu.SemaphoreType.DMA,
        ],
    )
    return k(feat, alpha, ids, ldst, ngrp)


def _decoder_body(hi_ref, hj_ref, t_ref, preds_ref, loss_ref):
    i = pl.program_id(0)
    j = pl.program_id(1)
    z = lax.dot_general(
        hi_ref[...], hj_ref[...], (((1,), (1,)), ((), ())),
        preferred_element_type=jnp.float32)
    p = 1.0 / (1.0 + jnp.exp(-z))
    logp = jnp.maximum(jnp.log(p), -100.0)
    log1mp = jnp.maximum(jnp.log(1.0 - p), -100.0)
    t = t_ref[...]
    contrib = -(t * logp + (1.0 - t) * log1mp)
    partial = jnp.sum(contrib, axis=(0, 1), keepdims=True) * (1.0 / (_N * _N))
    preds_ref[...] = p

    @pl.when((i == 0) & (j == 0))
    def _init():
        loss_ref[...] = jnp.zeros_like(loss_ref)

    loss_ref[...] += partial


def _decode(h, targets):
    grid = (_N // _BM, _N // _BN)
    preds, loss = pl.pallas_call(
        _decoder_body,
        grid=grid,
        in_specs=[
            pl.BlockSpec((_BM, _D), lambda i, j: (i, 0)),
            pl.BlockSpec((_BN, _D), lambda i, j: (j, 0)),
            pl.BlockSpec((_BM, _BN), lambda i, j: (i, j)),
        ],
        out_specs=[
            pl.BlockSpec((_BM, _BN), lambda i, j: (i, j)),
            pl.BlockSpec((1, 1), lambda i, j: (0, 0)),
        ],
        out_shape=[
            jax.ShapeDtypeStruct((_N, _N), jnp.float32),
            jax.ShapeDtypeStruct((1, 1), jnp.float32),
        ],
    )(h, h, targets)
    return loss[0, 0], preds


def _gs_body(feat_hbm, src_hbm, alpha_hbm, out_hbm,
             src_v, alpha_v, rows_v, sem):
    c = lax.axis_index("c")
    s = lax.axis_index("s")
    wid = s * _NC + c
    pltpu.sync_copy(src_hbm.at[wid], src_v)
    pltpu.sync_copy(alpha_hbm.at[wid], alpha_v)

    def _group(g, carry):
        pltpu.async_copy(feat_hbm.at[src_v.at[g]], rows_v, sem).wait()
        for kk in range(_KE // 16):
            a16 = alpha_v[g, pl.ds(kk * 16, 16)]
            for r in range(16):
                a = a16[r]
                row = kk * 16 + r
                for v in range(_D // 16):
                    rows_v[row, pl.ds(v * 16, 16)] = (
                        rows_v[row, pl.ds(v * 16, 16)] * a)
        pltpu.sync_copy(
            rows_v, out_hbm.at[pl.ds(wid * _EPT + g * _KE, _KE)])
        return carry

    lax.fori_loop(0, _EPT // _KE, _group, 0)


@jax.jit
def _sc_gather_scale(feat, src3, alpha3):
    k = pl.kernel(
        _gs_body,
        mesh=plsc.VectorSubcoreMesh(core_axis_name="c", subcore_axis_name="s"),
        out_type=jax.ShapeDtypeStruct((_E, _D), jnp.float32),
        scratch_types=[
            pltpu.VMEM((_EPT // _KE, _KE), jnp.int32),
            pltpu.VMEM((_EPT // _KE, _KE), jnp.float32),
            pltpu.VMEM((_KE, _D), jnp.float32),
            pltpu.SemaphoreType.DMA,
        ],
    )
    return k(feat, src3, alpha3)


_EPT_G = _EPT // _KE


def _edges3g(v):
    return v.reshape(_NW, _EPT_G, _KE)


def _wgcn_enc(x, src, dst, part, ew, W, b):
    scaled = _sc_gather_scale(x, _edges3g(src), _edges3g(ew))
    agg = jax.ops.segment_sum(scaled, dst, num_segments=_N)
    return jax.nn.relu(agg @ W + b)


def _gat_enc(x, src, dst, part, W, al, ar, bias):
    feat = (x @ W).reshape(_N, _H, -1)
    el = (feat * al[None, :, :]).sum(-1)
    er = (feat * ar[None, :, :]).sum(-1)
    e = jax.nn.leaky_relu(el[src] + er[dst], 0.2)
    emax = jax.ops.segment_max(e, dst, num_segments=_N)
    emax = jnp.where(jnp.isfinite(emax), emax, 0.0)
    ee = jnp.exp(e - emax[dst])
    den = jax.ops.segment_sum(ee, dst, num_segments=_N)
    alpha = ee / den[dst]
    feat_flat = feat.reshape(_N, _H * _D)
    outs = []
    src3g = _edges3g(src)
    for h in range(_H):
        scaled = _sc_gather_scale(feat_flat[:, h * _D:(h + 1) * _D],
                                  src3g, _edges3g(alpha[:, h]))
        outs.append(jax.ops.segment_sum(scaled, dst, num_segments=_N))
    out = jnp.stack(outs, axis=1)
    return out + bias.reshape(1, _H, -1)


def kernel(feature, edge_index_f, edge_index_b, e_f, e_b, targets, W_f, b_f,
           W_b, b_b, W_id, b_id, W1, b1, gat_W, gat_al, gat_ar, gat_b,
           gamma, beta):
    sf, df = edge_index_f[0], edge_index_f[1]
    sb, db = edge_index_b[0], edge_index_b[1]
    part_f = None
    part_b = None

    Hid = feature @ W_id + b_id
    fH = _wgcn_enc(feature, sf, df, part_f, e_f, W_f, b_f)
    bH = _wgcn_enc(feature, sb, db, part_b, e_b, W_b, b_b)
    hidden = Hid + fH + bH
    hidden = feature + hidden * jax.nn.sigmoid(feature)
    hidden = hidden @ W1 + b1

    idx = 0
    for _blk in range(_NRB):
        x = hidden
        for _lyr in range(2):
            of = _gat_enc(x, sf, df, part_f, gat_W[idx], gat_al[idx],
                          gat_ar[idx], gat_b[idx]); idx += 1
            ob = _gat_enc(x, sb, db, part_b, gat_W[idx], gat_al[idx],
                          gat_ar[idx], gat_b[idx]); idx += 1
            x = (of + ob).mean(1)
        hidden = x + hidden

    mu = hidden.mean(0)
    var = ((hidden - mu) ** 2).mean(0)
    h = (hidden - mu) / jnp.sqrt(var + 1e-5) * gamma + beta

    loss, preds = _decode(h, targets)
    return (loss, preds)


# speedup vs baseline: 3.6659x; 1.0076x over previous
import functools

import jax
import jax.numpy as jnp
from jax import lax
from jax.experimental import pallas as pl
from jax.experimental.pallas import tpu as pltpu
from jax.experimental.pallas import tpu_sc as plsc

_N = 4096
_D = 256
_H = 4
_NRB = 2
_E = 131072

_BM = 512
_BN = 512


_NC = 2
_NS = 16
_NW = _NC * _NS
_RPT = _N // _NW
_EPT = _E // _NW
_KE = 128
_CAPG = 66
_CAP = _CAPG * _KE
_CHUNK = 2048


def _part_body(dst_hbm, ids_hbm, ldst_hbm, ngrp_hbm,
               chunk_v, ids_v, ldst_v, n_v):
    c = lax.axis_index("c")
    s = lax.axis_index("s")
    wid = s * _NC + c
    lo = wid * _RPT
    iota = lax.iota(jnp.int32, 16)

    sent_v = jnp.full((16,), _RPT << 17, jnp.int32)

    def chunk_loop(i, cursor0):
        cursor0 = jnp.minimum(cursor0, _CAP - _KE - 2 * _CHUNK)
        pltpu.sync_copy(dst_hbm.at[pl.ds(i * _CHUNK, _CHUNK)], chunk_v)

        def vec_loop(v, cursor):
            d16 = chunk_v[pl.ds(v * 16, 16)]
            rel = d16 - lo
            m = (rel >= 0) & (rel < _RPT)
            mi = m.astype(jnp.int32)
            eid = iota + (i * _CHUNK + v * 16)
            pk = jnp.where(m, eid | (rel << 17), _RPT << 17)
            for r in range(16):
                ids_v[pl.ds(cursor, 16)] = jnp.full((16,), pk[r], jnp.int32)
                cursor = cursor + mi[r]
            return cursor

        return lax.fori_loop(0, _CHUNK // 16, vec_loop, cursor0)

    cursor = lax.fori_loop(0, _E // _CHUNK, chunk_loop, jnp.int32(0))
    for j in range((_KE // 16) + 2):
        ids_v[pl.ds(cursor + j * 16, 16)] = sent_v
    ng = (cursor + _KE - 1) // _KE
    ng = jnp.maximum(ng, 1)
    n_v[...] = jnp.broadcast_to(ng, (16,)).astype(jnp.int32)
    pltpu.sync_copy(n_v, ngrp_hbm.at[wid])

    def unpack(u, carry):
        pk = ids_v[pl.ds(u * 16, 16)]
        ldst_v[pl.ds(u * 16, 16)] = pk >> 17
        ids_v[pl.ds(u * 16, 16)] = pk & 0x1FFFF
        return carry

    lax.fori_loop(0, _CAP // 16, unpack, 0)
    pltpu.sync_copy(ids_v, ids_hbm.at[wid])
    pltpu.sync_copy(ldst_v, ldst_hbm.at[wid])


@jax.jit
def _sc_partition(dst):
    k = pl.kernel(
        _part_body,
        mesh=plsc.VectorSubcoreMesh(core_axis_name="c", subcore_axis_name="s"),
        out_type=[
            jax.ShapeDtypeStruct((_NW, _CAP), jnp.int32),
            jax.ShapeDtypeStruct((_NW, _CAP), jnp.int32),
            jax.ShapeDtypeStruct((_NW, 16), jnp.int32),
        ],
        scratch_types=[
            pltpu.VMEM((_CHUNK,), jnp.int32),
            pltpu.VMEM((_CAP,), jnp.int32),
            pltpu.VMEM((_CAP,), jnp.int32),
            pltpu.VMEM((16,), jnp.int32),
        ],
    )
    ids, ldst, ngrp = k(dst)
    return (ids.reshape(_NW, _CAPG, _KE), ldst.reshape(_NW, _CAPG, _KE), ngrp)


def _agg_body(feat_hbm, alpha_hbm, ids_hbm, ldst_hbm, ngrp_hbm, out_hbm,
              ids_v, ldst_v, alpha_g, rows_v, n_v, acc, sem):
    c = lax.axis_index("c")
    s = lax.axis_index("s")
    wid = s * _NC + c
    pltpu.sync_copy(ids_hbm.at[wid], ids_v)
    pltpu.sync_copy(ldst_hbm.at[wid], ldst_v)
    pltpu.sync_copy(ngrp_hbm.at[wid], n_v)
    ng = n_v[...][0]

    zero = jnp.zeros((16,), jnp.float32)

    def zrow(r, carry):
        for v in range(_D // 16):
            acc[r, pl.ds(v * 16, 16)] = zero
        return carry

    lax.fori_loop(0, _RPT + 8, zrow, 0)

    def group(g, carry):
        cp1 = pltpu.async_copy(feat_hbm.at[ids_v.at[g]], rows_v, sem)
        cp2 = pltpu.async_copy(alpha_hbm.at[ids_v.at[g]], alpha_g, sem)
        cp1.wait()
        cp2.wait()

        def sub(kk, carry2):
            a16 = alpha_g[pl.ds(kk * 16, 16)]
            l16 = ldst_v[g, pl.ds(kk * 16, 16)]
            rbase = kk * 16
            for r in range(16):
                a = a16[r]
                lr = l16[r]
                for v in range(_D // 16):
                    acc[lr, pl.ds(v * 16, 16)] = (
                        acc[lr, pl.ds(v * 16, 16)]
                        + rows_v[rbase + r, pl.ds(v * 16, 16)] * a)
            return carry2

        lax.fori_loop(0, _KE // 16, sub, 0)
        return carry

    lax.fori_loop(0, ng, group, 0)
    pltpu.sync_copy(acc.at[pl.ds(0, _RPT)],
                    out_hbm.at[pl.ds(wid * _RPT, _RPT)])


@jax.jit
def _sc_agg(feat, alpha, part):
    ids, ldst, ngrp = part
    k = pl.kernel(
        _agg_body,
        mesh=plsc.VectorSubcoreMesh(core_axis_name="c", subcore_axis_name="s"),
        out_type=jax.ShapeDtypeStruct((_N, _D), jnp.float32),
        scratch_types=[
            pltpu.VMEM((_CAPG, _KE), jnp.int32),
            pltpu.VMEM((_CAPG, _KE), jnp.int32),
            pltpu.VMEM((_KE,), jnp.float32),
            pltpu.VMEM((_KE, _D), jnp.float32),
            pltpu.VMEM((16,), jnp.int32),
            pltpu.VMEM((_RPT + 8, _D), jnp.float32),
            pltpu.SemaphoreType.DMA,
        ],
    )
    return k(feat, alpha, ids, ldst, ngrp)


def _decoder_body(hi_ref, hj_ref, t_ref, preds_ref, loss_ref):
    i = pl.program_id(0)
    j = pl.program_id(1)
    z = lax.dot_general(
        hi_ref[...], hj_ref[...], (((1,), (1,)), ((), ())),
        preferred_element_type=jnp.float32)
    p = 1.0 / (1.0 + jnp.exp(-z))
    logp = jnp.maximum(jnp.log(p), -100.0)
    log1mp = jnp.maximum(jnp.log(1.0 - p), -100.0)
    t = t_ref[...]
    contrib = -(t * logp + (1.0 - t) * log1mp)
    partial = jnp.sum(contrib, axis=(0, 1), keepdims=True) * (1.0 / (_N * _N))
    preds_ref[...] = p

    @pl.when((i == 0) & (j == 0))
    def _init():
        loss_ref[...] = jnp.zeros_like(loss_ref)

    loss_ref[...] += partial


def _decode(h, targets):
    grid = (_N // _BM, _N // _BN)
    preds, loss = pl.pallas_call(
        _decoder_body,
        grid=grid,
        in_specs=[
            pl.BlockSpec((_BM, _D), lambda i, j: (i, 0)),
            pl.BlockSpec((_BN, _D), lambda i, j: (j, 0)),
            pl.BlockSpec((_BM, _BN), lambda i, j: (i, j)),
        ],
        out_specs=[
            pl.BlockSpec((_BM, _BN), lambda i, j: (i, j)),
            pl.BlockSpec((1, 1), lambda i, j: (0, 0)),
        ],
        out_shape=[
            jax.ShapeDtypeStruct((_N, _N), jnp.float32),
            jax.ShapeDtypeStruct((1, 1), jnp.float32),
        ],
    )(h, h, targets)
    return loss[0, 0], preds


def _gs_body(feat_hbm, src_hbm, alpha_hbm, out_hbm,
             src_v, alpha_v, rows_v, sem):
    c = lax.axis_index("c")
    s = lax.axis_index("s")
    wid = s * _NC + c
    pltpu.sync_copy(src_hbm.at[wid], src_v)
    pltpu.sync_copy(alpha_hbm.at[wid], alpha_v)

    def _group(g, carry):
        pltpu.async_copy(feat_hbm.at[src_v.at[g]], rows_v, sem).wait()
        for kk in range(_KE // 16):
            a16 = alpha_v[g, pl.ds(kk * 16, 16)]
            for r in range(16):
                a = a16[r]
                row = kk * 16 + r
                for v in range(_D // 16):
                    rows_v[row, pl.ds(v * 16, 16)] = (
                        rows_v[row, pl.ds(v * 16, 16)] * a)
        pltpu.sync_copy(
            rows_v, out_hbm.at[pl.ds(wid * _EPT + g * _KE, _KE)])
        return carry

    lax.fori_loop(0, _EPT // _KE, _group, 0)


@jax.jit
def _sc_gather_scale(feat, src3, alpha3):
    k = pl.kernel(
        _gs_body,
        mesh=plsc.VectorSubcoreMesh(core_axis_name="c", subcore_axis_name="s"),
        out_type=jax.ShapeDtypeStruct((_E, _D), jnp.float32),
        scratch_types=[
            pltpu.VMEM((_EPT // _KE, _KE), jnp.int32),
            pltpu.VMEM((_EPT // _KE, _KE), jnp.float32),
            pltpu.VMEM((_KE, _D), jnp.float32),
            pltpu.SemaphoreType.DMA,
        ],
    )
    return k(feat, src3, alpha3)


_EPT_G = _EPT // _KE


def _edges3g(v):
    return v.reshape(_NW, _EPT_G, _KE)


def _wgcn_enc(x, src, dst, part, ew, W, b):
    scaled = _sc_gather_scale(x, _edges3g(src), _edges3g(ew))
    agg = jax.ops.segment_sum(scaled, dst, num_segments=_N)
    return jax.nn.relu(agg @ W + b)


def _gat_enc(x, src, dst, part, W, al, ar, bias):
    feat = (x @ W).reshape(_N, _H, -1)
    el = (feat * al[None, :, :]).sum(-1)
    er = (feat * ar[None, :, :]).sum(-1)
    e = jax.nn.leaky_relu(el[src] + er[dst], 0.2)
    emax = jax.ops.segment_max(e, dst, num_segments=_N)
    emax = jnp.where(jnp.isfinite(emax), emax, 0.0)
    ee = jnp.exp(e - emax[dst])
    den = jax.ops.segment_sum(ee, dst, num_segments=_N)
    alpha = ee / den[dst]
    feat_flat = feat.reshape(_N, _H * _D)
    outs = []
    src3g = _edges3g(src)
    for h in range(_H):
        scaled = _sc_gather_scale(feat_flat[:, h * _D:(h + 1) * _D],
                                  src3g, _edges3g(alpha[:, h]))
        outs.append(jax.ops.segment_sum(scaled, dst, num_segments=_N))
    out = jnp.stack(outs, axis=1)
    return out + bias.reshape(1, _H, -1)


def kernel(feature, edge_index_f, edge_index_b, e_f, e_b, targets, W_f, b_f,
           W_b, b_b, W_id, b_id, W1, b1, gat_W, gat_al, gat_ar, gat_b,
           gamma, beta):
    sf, df = edge_index_f[0], edge_index_f[1]
    sb, db = edge_index_b[0], edge_index_b[1]
    part_f = None
    part_b = None

    Hid = feature @ W_id + b_id
    fH = _wgcn_enc(feature, sf, df, part_f, e_f, W_f, b_f)
    bH = _wgcn_enc(feature, sb, db, part_b, e_b, W_b, b_b)
    hidden = Hid + fH + bH
    hidden = feature + hidden * jax.nn.sigmoid(feature)
    hidden = hidden @ W1 + b1

    idx = 0
    for _blk in range(_NRB):
        x = hidden
        for _lyr in range(2):
            of = _gat_enc(x, sf, df, part_f, gat_W[idx], gat_al[idx],
                          gat_ar[idx], gat_b[idx]); idx += 1
            ob = _gat_enc(x, sb, db, part_b, gat_W[idx], gat_al[idx],
                          gat_ar[idx], gat_b[idx]); idx += 1
            x = (of + ob).mean(1)
        hidden = x + hidden

    mu = hidden.mean(0)
    var = ((hidden - mu) ** 2).mean(0)
    h = (hidden - mu) / jnp.sqrt(var + 1e-5) * gamma + beta

    loss, preds = _decode(h, targets)
    return (loss, preds)
